# Initial kernel scaffold; baseline (speedup 1.0000x reference)
import jax
import jax.numpy as jnp
from jax.experimental import pallas as pl


def kernel(fm0, fm1, fm2, cell_anchors):
    raise NotImplementedError("write your pallas kernel here")


# speedup vs baseline: 3.5373x; 3.5373x over previous
import jax
import jax.numpy as jnp
from jax.experimental import pallas as pl
from jax.experimental.pallas import tpu as pltpu

_N = 16
_C = 255
_ROWS = 1444


def _act(x):
    c = jax.lax.broadcasted_iota(jnp.int32, x.shape, 0) % 85
    raw = (c == 2) | (c == 3)
    return jnp.where(raw, x, jax.nn.sigmoid(x))


def _body(a_ref, b_ref, c_ref, o_ref):
    j = pl.program_id(1)

    for jj in range(4):
        @pl.when(j == jj)
        def _(jj=jj):
            y = _act(a_ref[0, :, jj * 1444:(jj + 1) * 1444]).T
            o_ref[0] = y.reshape(_ROWS, 3, 85)

    @pl.when(j == 4)
    def _():
        y = _act(b_ref[0]).T
        o_ref[0] = y.reshape(_ROWS, 3, 85)

    @pl.when(j == 5)
    def _():
        y = _act(c_ref[0]).T
        o_ref[0, :361] = y.reshape(361, 3, 85)


def kernel(fm0, fm1, fm2, cell_anchors):
    del cell_anchors
    a = fm0.reshape(_N, _C, 5776)
    b = fm1.reshape(_N, _C, 1444)
    c = fm2.reshape(_N, _C, 361)
    out = pl.pallas_call(
        _body,
        grid=(_N, 6),
        in_specs=[
            pl.BlockSpec((1, _C, 5776), lambda n, j: (n, 0, 0)),
            pl.BlockSpec((1, _C, 1444), lambda n, j: (n, 0, 0)),
            pl.BlockSpec((1, _C, 361), lambda n, j: (n, 0, 0)),
        ],
        out_specs=pl.BlockSpec((1, _ROWS, 3, 85), lambda n, j: (n, j, 0, 0)),
        out_shape=jax.ShapeDtypeStruct((_N, 7581, 3, 85), jnp.float32),
    )(a, b, c)
    return out


# speedup vs baseline: 5.6533x; 1.5982x over previous
import jax
import jax.numpy as jnp
from jax.experimental import pallas as pl
from jax.experimental.pallas import tpu as pltpu

_N = 16
_C = 255


def _act(x):
    c = jax.lax.broadcasted_iota(jnp.int32, x.shape, 0) % 85
    raw = (c == 2) | (c == 3)
    return jnp.where(raw, x, jax.nn.sigmoid(x))


def _body(a_ref, b_ref, c_ref, o_ref):
    o_ref[0, 0:5776, :] = _act(a_ref[0]).T
    o_ref[0, 5776:7220, :] = _act(b_ref[0]).T
    o_ref[0, 7220:7581, :] = _act(c_ref[0]).T


def kernel(fm0, fm1, fm2, cell_anchors):
    del cell_anchors
    a = fm0.reshape(_N, _C, 5776)
    b = fm1.reshape(_N, _C, 1444)
    c = fm2.reshape(_N, _C, 361)
    out = pl.pallas_call(
        _body,
        grid=(_N,),
        in_specs=[
            pl.BlockSpec((1, _C, 5776), lambda n: (n, 0, 0)),
            pl.BlockSpec((1, _C, 1444), lambda n: (n, 0, 0)),
            pl.BlockSpec((1, _C, 361), lambda n: (n, 0, 0)),
        ],
        out_specs=pl.BlockSpec((1, 7581, _C), lambda n: (n, 0, 0)),
        out_shape=jax.ShapeDtypeStruct((_N, 7581, _C), jnp.float32),
    )(a, b, c)
    return out.reshape(_N, 7581, 3, 85)
